# baseline (speedup 1.0000x reference)
import functools

import jax
import jax.numpy as jnp
from jax.experimental import pallas as pl
from jax.experimental.pallas import tpu as pltpu

_A = 9
_G = 16


def _rpn_body(H, W, flat_ref, w1_ref, b1_ref, w2_ref, b2_ref,
              wcat_ref, bcat_ref, out_ref, xb_ref):
    C = flat_ref.shape[2]
    XQ = (H + 2) * W
    NP = H * W
    wpos = jax.lax.broadcasted_iota(jnp.int32, (XQ, 1), 0) % W
    zero = jnp.zeros((), dtype=jnp.bfloat16)

    xb_ref[pl.ds(_G, XQ), C:2 * C] = flat_ref[0, pl.ds(_G, XQ), :]
    left = flat_ref[0, pl.ds(_G - 1, XQ), :]
    xb_ref[pl.ds(_G, XQ), 0:C] = jnp.where(wpos == 0, zero, left)
    right = flat_ref[0, pl.ds(_G + 1, XQ), :]
    xb_ref[pl.ds(_G, XQ), 2 * C:3 * C] = jnp.where(wpos == W - 1, zero, right)

    acc = jnp.dot(xb_ref[pl.ds(_G, NP), :], w1_ref[0],
                  preferred_element_type=jnp.float32)
    acc += jnp.dot(xb_ref[pl.ds(_G + W, NP), :], w1_ref[1],
                   preferred_element_type=jnp.float32)
    acc += jnp.dot(xb_ref[pl.ds(_G + 2 * W, NP), :], w1_ref[2],
                   preferred_element_type=jnp.float32)
    h1 = jnp.maximum(acc + b1_ref[0], 0.0).astype(jnp.bfloat16)

    xb_ref[pl.ds(0, _G + W), C:2 * C] = jnp.zeros((_G + W, C),
                                                  dtype=jnp.bfloat16)
    xb_ref[pl.ds(_G + W + NP, _G + W), C:2 * C] = jnp.zeros(
        (_G + W, C), dtype=jnp.bfloat16)
    xb_ref[pl.ds(_G + W, NP), C:2 * C] = h1

    left = xb_ref[pl.ds(_G - 1, XQ), C:2 * C]
    xb_ref[pl.ds(_G, XQ), 0:C] = jnp.where(wpos == 0, zero, left)
    right = xb_ref[pl.ds(_G + 1, XQ), C:2 * C]
    xb_ref[pl.ds(_G, XQ), 2 * C:3 * C] = jnp.where(wpos == W - 1, zero, right)

    acc2 = jnp.dot(xb_ref[pl.ds(_G, NP), :], w2_ref[0],
                   preferred_element_type=jnp.float32)
    acc2 += jnp.dot(xb_ref[pl.ds(_G + W, NP), :], w2_ref[1],
                    preferred_element_type=jnp.float32)
    acc2 += jnp.dot(xb_ref[pl.ds(_G + 2 * W, NP), :], w2_ref[2],
                    preferred_element_type=jnp.float32)
    h2 = jnp.maximum(acc2 + b2_ref[0], 0.0).astype(jnp.bfloat16)

    z = jnp.dot(h2, wcat_ref[...], preferred_element_type=jnp.float32)
    z = z + bcat_ref[0]
    ch = jax.lax.broadcasted_iota(jnp.int32, (1, 6 * _A), 1) % 6
    out_ref[0] = jnp.where(ch < 2, 1.0 / (1.0 + jnp.exp(-z)), z)


def kernel(input, W1, b1, W2, b2, Wreg, breg, Wcls, bcls):
    B, H, W, C = input.shape
    A = _A
    NP = H * W
    XQ = (H + 2) * W
    TOT = XQ + 2 * _G

    xp = jnp.pad(input, ((0, 0), (1, 1), (0, 0), (0, 0)))
    flat = jnp.pad(xp.reshape(B, XQ, C), ((0, 0), (_G, _G), (0, 0)))
    flat = flat.astype(jnp.bfloat16)

    W1f = W1.reshape(3, 3 * C, C).astype(jnp.bfloat16)
    W2f = W2.reshape(3, 3 * C, C).astype(jnp.bfloat16)

    Wc = Wcls.reshape(C, A, 2)
    d0 = (Wc[:, :, 0] - Wc[:, :, 1])[:, :, None]
    Wcat = jnp.concatenate([d0, -d0, Wreg.reshape(C, A, 4)],
                           axis=2).reshape(C, 6 * A).astype(jnp.bfloat16)
    bc = bcls.reshape(A, 2)
    bd0 = (bc[:, 0] - bc[:, 1])[:, None]
    bcat = jnp.concatenate([bd0, -bd0, breg.reshape(A, 4)],
                           axis=1).reshape(1, 6 * A)

    out = pl.pallas_call(
        functools.partial(_rpn_body, H, W),
        grid=(B,),
        in_specs=[
            pl.BlockSpec((1, TOT, C), lambda b: (b, 0, 0)),
            pl.BlockSpec((3, 3 * C, C), lambda b: (0, 0, 0)),
            pl.BlockSpec((1, C), lambda b: (0, 0)),
            pl.BlockSpec((3, 3 * C, C), lambda b: (0, 0, 0)),
            pl.BlockSpec((1, C), lambda b: (0, 0)),
            pl.BlockSpec((C, 6 * A), lambda b: (0, 0)),
            pl.BlockSpec((1, 6 * A), lambda b: (0, 0)),
        ],
        out_specs=pl.BlockSpec((1, NP, 6 * A), lambda b: (b, 0, 0)),
        out_shape=jax.ShapeDtypeStruct((B, NP, 6 * A), jnp.float32),
        scratch_shapes=[
            pltpu.VMEM((TOT, 3 * C), jnp.bfloat16),
        ],
    )(flat, W1f, b1.reshape(1, C), W2f, b2.reshape(1, C), Wcat, bcat)

    return out.reshape(B, H, W, A, 6)


# speedup vs baseline: 1.1907x; 1.1907x over previous
import functools

import jax
import jax.numpy as jnp
from jax.experimental import pallas as pl
from jax.experimental.pallas import tpu as pltpu

_A = 9
_G = 16


def _rpn_body(H, W, x_ref, w1_ref, b1_ref, w2_ref, b2_ref,
              wreg_ref, breg_ref, wcls_ref, bcls_ref, out_ref, xb_ref):
    C = x_ref.shape[2]
    A = _A
    XQ = (H + 2) * W
    NP = H * W
    wpos = jax.lax.broadcasted_iota(jnp.int32, (XQ, 1), 0) % W
    zero = jnp.zeros((), dtype=jnp.bfloat16)

    def build_taps():
        left = xb_ref[pl.ds(_G - 1, XQ), C:2 * C]
        xb_ref[pl.ds(_G, XQ), 0:C] = jnp.where(wpos == 0, zero, left)
        right = xb_ref[pl.ds(_G + 1, XQ), C:2 * C]
        xb_ref[pl.ds(_G, XQ), 2 * C:3 * C] = jnp.where(wpos == W - 1, zero,
                                                       right)

    def conv(w_ref, b_ref):
        w = w_ref[...].astype(jnp.bfloat16)
        acc = jnp.dot(xb_ref[pl.ds(_G, NP), :], w[0],
                      preferred_element_type=jnp.float32)
        acc += jnp.dot(xb_ref[pl.ds(_G + W, NP), :], w[1],
                       preferred_element_type=jnp.float32)
        acc += jnp.dot(xb_ref[pl.ds(_G + 2 * W, NP), :], w[2],
                       preferred_element_type=jnp.float32)
        return jnp.maximum(acc + b_ref[0], 0.0).astype(jnp.bfloat16)

    def stage_center(img_bf16):
        xb_ref[pl.ds(0, _G + W), C:2 * C] = jnp.zeros((_G + W, C),
                                                      dtype=jnp.bfloat16)
        xb_ref[pl.ds(_G + W + NP, _G + W), C:2 * C] = jnp.zeros(
            (_G + W, C), dtype=jnp.bfloat16)
        xb_ref[pl.ds(_G + W, NP), C:2 * C] = img_bf16

    stage_center(x_ref[0].astype(jnp.bfloat16))
    build_taps()
    h1 = conv(w1_ref, b1_ref)

    stage_center(h1)
    build_taps()
    h2 = conv(w2_ref, b2_ref)

    K6 = 6 * A
    r = jax.lax.broadcasted_iota(jnp.int32, (K6, K6), 0)
    c = jax.lax.broadcasted_iota(jnp.int32, (K6, K6), 1)
    a = c // 6
    j = c % 6
    plus = ((j >= 2) & (r == 2 * A + 4 * a + j - 2)) \
        | ((j == 0) & (r == 2 * a)) | ((j == 1) & (r == 2 * a + 1))
    minus = ((j == 0) & (r == 2 * a + 1)) | ((j == 1) & (r == 2 * a))
    P = jnp.where(plus, 1.0, 0.0) - jnp.where(minus, 1.0, 0.0)

    whead = jnp.concatenate([wcls_ref[...], wreg_ref[...]], axis=1)
    wcat = jnp.dot(whead, P,
                   preferred_element_type=jnp.float32).astype(jnp.bfloat16)
    bvec = jnp.concatenate([bcls_ref[...], breg_ref[...]], axis=1)
    bcat = jnp.dot(bvec, P, preferred_element_type=jnp.float32)

    z = jnp.dot(h2, wcat, preferred_element_type=jnp.float32) + bcat
    ch = jax.lax.broadcasted_iota(jnp.int32, (1, 6 * A), 1) % 6
    out_ref[0] = jnp.where(ch < 2, 1.0 / (1.0 + jnp.exp(-z)), z)


def kernel(input, W1, b1, W2, b2, Wreg, breg, Wcls, bcls):
    B, H, W, C = input.shape
    A = _A
    NP = H * W
    TOT = (H + 2) * W + 2 * _G

    x2d = input.reshape(B, NP, C)
    W1r = W1.reshape(3, 3 * C, C)
    W2r = W2.reshape(3, 3 * C, C)

    out = pl.pallas_call(
        functools.partial(_rpn_body, H, W),
        grid=(B,),
        in_specs=[
            pl.BlockSpec((1, NP, C), lambda b: (b, 0, 0)),
            pl.BlockSpec((3, 3 * C, C), lambda b: (0, 0, 0)),
            pl.BlockSpec((1, C), lambda b: (0, 0)),
            pl.BlockSpec((3, 3 * C, C), lambda b: (0, 0, 0)),
            pl.BlockSpec((1, C), lambda b: (0, 0)),
            pl.BlockSpec((C, 4 * A), lambda b: (0, 0)),
            pl.BlockSpec((1, 4 * A), lambda b: (0, 0)),
            pl.BlockSpec((C, 2 * A), lambda b: (0, 0)),
            pl.BlockSpec((1, 2 * A), lambda b: (0, 0)),
        ],
        out_specs=pl.BlockSpec((1, NP, 6 * A), lambda b: (b, 0, 0)),
        out_shape=jax.ShapeDtypeStruct((B, NP, 6 * A), jnp.float32),
        scratch_shapes=[
            pltpu.VMEM((TOT, 3 * C), jnp.bfloat16),
        ],
    )(x2d, W1r, b1.reshape(1, C), W2r, b2.reshape(1, C),
      Wreg.reshape(C, 4 * A), breg.reshape(1, 4 * A),
      Wcls.reshape(C, 2 * A), bcls.reshape(1, 2 * A))

    return out.reshape(B, H, W, A, 6)


# speedup vs baseline: 1.5003x; 1.2600x over previous
import functools

import jax
import jax.numpy as jnp
from jax.experimental import pallas as pl
from jax.experimental.pallas import tpu as pltpu

_A = 9
_G = 16
_NK = 4


def _rpn_body(H, W, x_ref, w1_ref, b1_ref, w2_ref, b2_ref,
              wreg_ref, breg_ref, wcls_ref, bcls_ref, out_ref,
              xb_ref, yb_ref):
    C = x_ref.shape[2]
    A = _A
    XQ = (H + 2) * W
    NP = H * W
    MC = NP // _NK
    zero = jnp.zeros((), dtype=jnp.bfloat16)

    def zero_bands(buf):
        buf[pl.ds(0, _G + W), C:2 * C] = jnp.zeros((_G + W, C),
                                                   dtype=jnp.bfloat16)
        buf[pl.ds(_G + W + NP, _G + W), C:2 * C] = jnp.zeros(
            (_G + W, C), dtype=jnp.bfloat16)

    def stage(k):
        xb_ref[pl.ds(_G + W + k * MC, MC), C:2 * C] = \
            x_ref[0, pl.ds(k * MC, MC), :].astype(jnp.bfloat16)

    def taps(buf, k):
        t0 = _G if k == 0 else _G + 2 * W + k * MC
        ln = MC + 2 * W if k == 0 else MC
        wpos = jax.lax.broadcasted_iota(jnp.int32, (ln, 1), 0) + (t0 - _G)
        wpos = wpos % W
        left = buf[pl.ds(t0 - 1, ln), C:2 * C]
        buf[pl.ds(t0, ln), 0:C] = jnp.where(wpos == 0, zero, left)
        right = buf[pl.ds(t0 + 1, ln), C:2 * C]
        buf[pl.ds(t0, ln), 2 * C:3 * C] = jnp.where(wpos == W - 1, zero,
                                                    right)

    def conv_chunk(buf, w, b_ref, k):
        acc = jnp.dot(buf[pl.ds(_G + k * MC, MC), :], w[0],
                      preferred_element_type=jnp.float32)
        acc += jnp.dot(buf[pl.ds(_G + W + k * MC, MC), :], w[1],
                       preferred_element_type=jnp.float32)
        acc += jnp.dot(buf[pl.ds(_G + 2 * W + k * MC, MC), :], w[2],
                       preferred_element_type=jnp.float32)
        return jnp.maximum(acc + b_ref[0], 0.0).astype(jnp.bfloat16)

    def c1(k):
        yb_ref[pl.ds(_G + W + k * MC, MC), C:2 * C] = \
            conv_chunk(xb_ref, w1c, b1_ref, k)

    def head(k, h2k):
        z = jnp.dot(h2k, wcat, preferred_element_type=jnp.float32) + bcat
        ch = jax.lax.broadcasted_iota(jnp.int32, (1, 6 * A), 1) % 6
        out_ref[0, pl.ds(k * MC, MC), :] = jnp.where(
            ch < 2, 1.0 / (1.0 + jnp.exp(-z)), z)

    w1c = w1_ref[...].astype(jnp.bfloat16)
    w2c = w2_ref[...].astype(jnp.bfloat16)

    K6 = 6 * A
    r = jax.lax.broadcasted_iota(jnp.int32, (K6, K6), 0)
    c = jax.lax.broadcasted_iota(jnp.int32, (K6, K6), 1)
    a6 = c // 6
    j6 = c % 6
    plus = ((j6 >= 2) & (r == 2 * A + 4 * a6 + j6 - 2)) \
        | ((j6 == 0) & (r == 2 * a6)) | ((j6 == 1) & (r == 2 * a6 + 1))
    minus = ((j6 == 0) & (r == 2 * a6 + 1)) | ((j6 == 1) & (r == 2 * a6))
    P = jnp.where(plus, 1.0, 0.0) - jnp.where(minus, 1.0, 0.0)
    whead = jnp.concatenate([wcls_ref[...], wreg_ref[...]], axis=1)
    wcat = jnp.dot(whead, P,
                   preferred_element_type=jnp.float32).astype(jnp.bfloat16)
    bvec = jnp.concatenate([bcls_ref[...], breg_ref[...]], axis=1)
    bcat = jnp.dot(bvec, P, preferred_element_type=jnp.float32)

    zero_bands(xb_ref)
    zero_bands(yb_ref)

    stage(0)
    stage(1)
    taps(xb_ref, 0)
    c1(0)
    stage(2)
    taps(xb_ref, 1)
    c1(1)
    taps(yb_ref, 0)
    stage(3)
    taps(xb_ref, 2)
    c1(2)
    taps(yb_ref, 1)
    h2_0 = conv_chunk(yb_ref, w2c, b2_ref, 0)
    head(0, h2_0)
    taps(xb_ref, 3)
    c1(3)
    taps(yb_ref, 2)
    h2_1 = conv_chunk(yb_ref, w2c, b2_ref, 1)
    head(1, h2_1)
    taps(yb_ref, 3)
    h2_2 = conv_chunk(yb_ref, w2c, b2_ref, 2)
    head(2, h2_2)
    h2_3 = conv_chunk(yb_ref, w2c, b2_ref, 3)
    head(3, h2_3)


def kernel(input, W1, b1, W2, b2, Wreg, breg, Wcls, bcls):
    B, H, W, C = input.shape
    A = _A
    NP = H * W
    TOT = (H + 2) * W + 2 * _G

    x2d = input.reshape(B, NP, C)
    W1r = W1.reshape(3, 3 * C, C)
    W2r = W2.reshape(3, 3 * C, C)

    out = pl.pallas_call(
        functools.partial(_rpn_body, H, W),
        grid=(B,),
        in_specs=[
            pl.BlockSpec((1, NP, C), lambda b: (b, 0, 0)),
            pl.BlockSpec((3, 3 * C, C), lambda b: (0, 0, 0)),
            pl.BlockSpec((1, C), lambda b: (0, 0)),
            pl.BlockSpec((3, 3 * C, C), lambda b: (0, 0, 0)),
            pl.BlockSpec((1, C), lambda b: (0, 0)),
            pl.BlockSpec((C, 4 * A), lambda b: (0, 0)),
            pl.BlockSpec((1, 4 * A), lambda b: (0, 0)),
            pl.BlockSpec((C, 2 * A), lambda b: (0, 0)),
            pl.BlockSpec((1, 2 * A), lambda b: (0, 0)),
        ],
        out_specs=pl.BlockSpec((1, NP, 6 * A), lambda b: (b, 0, 0)),
        out_shape=jax.ShapeDtypeStruct((B, NP, 6 * A), jnp.float32),
        scratch_shapes=[
            pltpu.VMEM((TOT, 3 * C), jnp.bfloat16),
            pltpu.VMEM((TOT, 3 * C), jnp.bfloat16),
        ],
    )(x2d, W1r, b1.reshape(1, C), W2r, b2.reshape(1, C),
      Wreg.reshape(C, 4 * A), breg.reshape(1, 4 * A),
      Wcls.reshape(C, 2 * A), bcls.reshape(1, 2 * A))

    return out.reshape(B, H, W, A, 6)


# speedup vs baseline: 1.5923x; 1.0613x over previous
import functools

import jax
import jax.numpy as jnp
from jax.experimental import pallas as pl
from jax.experimental.pallas import tpu as pltpu

_A = 9
_G = 16
_NK = 8


def _rpn_body(H, W, x_ref, w1_ref, b1_ref, w2_ref, b2_ref,
              wreg_ref, breg_ref, wcls_ref, bcls_ref, out_ref,
              xb_ref, yb_ref):
    C = x_ref.shape[2]
    A = _A
    XQ = (H + 2) * W
    NP = H * W
    MC = NP // _NK
    zero = jnp.zeros((), dtype=jnp.bfloat16)

    def zero_bands(buf):
        buf[pl.ds(0, _G + W), C:2 * C] = jnp.zeros((_G + W, C),
                                                   dtype=jnp.bfloat16)
        buf[pl.ds(_G + W + NP, _G + W), C:2 * C] = jnp.zeros(
            (_G + W, C), dtype=jnp.bfloat16)

    def stage(k):
        xb_ref[pl.ds(_G + W + k * MC, MC), C:2 * C] = \
            x_ref[0, pl.ds(k * MC, MC), :].astype(jnp.bfloat16)

    def taps(buf, k):
        t0 = _G if k == 0 else _G + 2 * W + k * MC
        ln = MC + 2 * W if k == 0 else MC
        wpos = jax.lax.broadcasted_iota(jnp.int32, (ln, 1), 0) + (t0 - _G)
        wpos = wpos % W
        left = buf[pl.ds(t0 - 1, ln), C:2 * C]
        buf[pl.ds(t0, ln), 0:C] = jnp.where(wpos == 0, zero, left)
        right = buf[pl.ds(t0 + 1, ln), C:2 * C]
        buf[pl.ds(t0, ln), 2 * C:3 * C] = jnp.where(wpos == W - 1, zero,
                                                    right)

    def conv_chunk(buf, w, b_ref, k):
        acc = jnp.dot(buf[pl.ds(_G + k * MC, MC), :], w[0],
                      preferred_element_type=jnp.float32)
        acc += jnp.dot(buf[pl.ds(_G + W + k * MC, MC), :], w[1],
                       preferred_element_type=jnp.float32)
        acc += jnp.dot(buf[pl.ds(_G + 2 * W + k * MC, MC), :], w[2],
                       preferred_element_type=jnp.float32)
        return jnp.maximum(acc + b_ref[0], 0.0).astype(jnp.bfloat16)

    def c1(k):
        yb_ref[pl.ds(_G + W + k * MC, MC), C:2 * C] = \
            conv_chunk(xb_ref, w1c, b1_ref, k)

    def head(k, h2k):
        z = jnp.dot(h2k, wcat, preferred_element_type=jnp.float32) + bcat
        ch = jax.lax.broadcasted_iota(jnp.int32, (1, 6 * A), 1) % 6
        out_ref[0, pl.ds(k * MC, MC), :] = jnp.where(
            ch < 2, 1.0 / (1.0 + jnp.exp(-z)), z)

    w1c = w1_ref[...].astype(jnp.bfloat16)
    w2c = w2_ref[...].astype(jnp.bfloat16)

    K6 = 6 * A
    r = jax.lax.broadcasted_iota(jnp.int32, (K6, K6), 0)
    c = jax.lax.broadcasted_iota(jnp.int32, (K6, K6), 1)
    a6 = c // 6
    j6 = c % 6
    plus = ((j6 >= 2) & (r == 2 * A + 4 * a6 + j6 - 2)) \
        | ((j6 == 0) & (r == 2 * a6)) | ((j6 == 1) & (r == 2 * a6 + 1))
    minus = ((j6 == 0) & (r == 2 * a6 + 1)) | ((j6 == 1) & (r == 2 * a6))
    P = jnp.where(plus, 1.0, 0.0) - jnp.where(minus, 1.0, 0.0)
    whead = jnp.concatenate([wcls_ref[...], wreg_ref[...]], axis=1)
    wcat = jnp.dot(whead, P,
                   preferred_element_type=jnp.float32).astype(jnp.bfloat16)
    bvec = jnp.concatenate([bcls_ref[...], breg_ref[...]], axis=1)
    bcat = jnp.dot(bvec, P, preferred_element_type=jnp.float32)

    zero_bands(xb_ref)
    zero_bands(yb_ref)

    def c2h(k):
        head(k, conv_chunk(yb_ref, w2c, b2_ref, k))

    stage(0)
    stage(1)
    for k in range(_NK):
        if 1 <= k and k + 1 < _NK:
            stage(k + 1)
        taps(xb_ref, k)
        c1(k)
        if k >= 1:
            taps(yb_ref, k - 1)
        if k >= 2:
            c2h(k - 2)
    taps(yb_ref, _NK - 1)
    c2h(_NK - 2)
    c2h(_NK - 1)


def kernel(input, W1, b1, W2, b2, Wreg, breg, Wcls, bcls):
    B, H, W, C = input.shape
    A = _A
    NP = H * W
    TOT = (H + 2) * W + 2 * _G

    x2d = input.reshape(B, NP, C)
    W1r = W1.reshape(3, 3 * C, C)
    W2r = W2.reshape(3, 3 * C, C)

    out = pl.pallas_call(
        functools.partial(_rpn_body, H, W),
        grid=(B,),
        in_specs=[
            pl.BlockSpec((1, NP, C), lambda b: (b, 0, 0)),
            pl.BlockSpec((3, 3 * C, C), lambda b: (0, 0, 0)),
            pl.BlockSpec((1, C), lambda b: (0, 0)),
            pl.BlockSpec((3, 3 * C, C), lambda b: (0, 0, 0)),
            pl.BlockSpec((1, C), lambda b: (0, 0)),
            pl.BlockSpec((C, 4 * A), lambda b: (0, 0)),
            pl.BlockSpec((1, 4 * A), lambda b: (0, 0)),
            pl.BlockSpec((C, 2 * A), lambda b: (0, 0)),
            pl.BlockSpec((1, 2 * A), lambda b: (0, 0)),
        ],
        out_specs=pl.BlockSpec((1, NP, 6 * A), lambda b: (b, 0, 0)),
        out_shape=jax.ShapeDtypeStruct((B, NP, 6 * A), jnp.float32),
        scratch_shapes=[
            pltpu.VMEM((TOT, 3 * C), jnp.bfloat16),
            pltpu.VMEM((TOT, 3 * C), jnp.bfloat16),
        ],
    )(x2d, W1r, b1.reshape(1, C), W2r, b2.reshape(1, C),
      Wreg.reshape(C, 4 * A), breg.reshape(1, 4 * A),
      Wcls.reshape(C, 2 * A), bcls.reshape(1, 2 * A))

    return out.reshape(B, H, W, A, 6)
